# baseline (speedup 1.0000x reference)
import functools

import jax
import jax.numpy as jnp
from jax import lax
from jax.experimental import pallas as pl
from jax.experimental.pallas import tpu as pltpu
from jax.experimental.pallas import tpu_sc as plsc

L = 16
D = 8
NC = 2
NS = 16
NW = NC * NS
LANE = 128


def _build_call(n):
    npad = -(-n // LANE) * LANE
    ntiles = npad // LANE
    tbase_w = ntiles // NW
    textra = ntiles - tbase_w * NW
    cols_hi = (tbase_w + 1) * LANE
    cols_lo = tbase_w * LANE
    shift_max = npad - n + LANE
    cols_buf = cols_hi + shift_max

    win = 2 * cols_hi
    win_buf = win + (npad - n)
    assert tbase_w % 2 == 0 and LANE // L == 8 and win // L % 2 == 0

    mesh = plsc.VectorSubcoreMesh(core_axis_name="c", subcore_axis_name="s")

    @functools.partial(
        pl.kernel,
        out_type=jax.ShapeDtypeStruct((D, n), jnp.float32),
        mesh=mesh,
        scratch_types=[
            pltpu.VMEM((win_buf,), jnp.int32),
            pltpu.VMEM((cols_buf,), jnp.float32),
            pltpu.VMEM((D, cols_hi), jnp.float32),
            pltpu.VMEM((L,), jnp.int32),
            pltpu.VMEM((NS * L,), jnp.int32),
            pltpu.VMEM_SHARED((NS * L,), jnp.int32),
            pltpu.SemaphoreType.DMA,
            pltpu.SemaphoreType.DMA,
            pltpu.SemaphoreType.DMA,
        ],
        compiler_params=pltpu.CompilerParams(needs_layout_passes=False),
    )
    def call(ls_hbm, sv_hbm, out_hbm,
             vals2_v, src_v, out2_v,
             minvec_v, allmins_v, mins_sh, sem_v, sem_s, sem_o):
        c = lax.axis_index("c")
        s = lax.axis_index("s")
        w = s * NC + c

        tb = tbase_w * w + jnp.minimum(w, textra)
        cb = tb * LANE
        base_s = jnp.minimum((tbase_w * (2 * s)
                              + jnp.minimum(2 * s, textra)) * LANE, n - win)
        shift2 = cb - base_s
        base2 = jnp.minimum(cb, n - cols_hi)
        shift = cb - base2
        hi = w < textra
        ngc2 = (tbase_w + jnp.where(hi, 1, 0)) * (LANE // L // 2)

        h_v = pltpu.async_copy(ls_hbm.at[pl.ds(base_s, win)],
                               vals2_v.at[pl.ds(0, win)], sem_v)
        h_s = pltpu.async_copy(sv_hbm.at[pl.ds(base2, cols_hi)],
                               src_v.at[pl.ds(0, cols_hi)], sem_s)

        h_v.wait()

        def mstep(i, m):
            m = jnp.minimum(m, vals2_v[pl.ds(i * (2 * L), L)])
            return jnp.minimum(m, vals2_v[pl.ds(i * (2 * L) + L, L)])
        m = lax.fori_loop(0, win // L // 2, mstep,
                          jnp.full((L,), jnp.iinfo(jnp.int32).max,
                                   jnp.int32))
        minvec_v[...] = m
        pltpu.sync_copy(minvec_v, mins_sh.at[pl.ds(s * L, L)])
        plsc.subcore_barrier()
        pltpu.sync_copy(mins_sh, allmins_v)

        def mstep2(i, m):
            return jnp.minimum(m, allmins_v[pl.ds(i * L, L)])
        mall = lax.fori_loop(0, NS, mstep2,
                             jnp.full((L,), jnp.iinfo(jnp.int32).max,
                                      jnp.int32))
        gmin = mall[0]
        for j in range(1, L):
            gmin = jnp.minimum(gmin, mall[j])

        h_s.wait()
        fzero = jnp.zeros((L,), jnp.float32)
        lane = lax.iota(jnp.int32, L)

        def group(g):
            v = vals2_v[pl.ds(shift2 + g * L, L)] - gmin
            v = jnp.minimum(v, D - 1)
            v = jnp.maximum(v, 0)
            sv = src_v[pl.ds(shift + g * L, L)]
            for d in range(D):
                out2_v[d, pl.ds(g * L, L)] = fzero
            plsc.store_scatter(out2_v, [v, g * L + lane], sv)

        def wstep(i, carry):
            group(2 * i)
            group(2 * i + 1)
            return carry

        ghalf = cols_lo // L // 2
        lax.fori_loop(0, ghalf // 2, wstep, 0)
        h_o = pltpu.async_copy(out2_v.at[:, pl.ds(0, cols_lo // 2)],
                               out_hbm.at[:, pl.ds(cb, cols_lo // 2)],
                               sem_o)
        lax.fori_loop(ghalf // 2, ngc2, wstep, 0)

        last = w >= NW - 1
        half = cols_lo // 2
        cols_last = n - (ntiles - tbase_w) * LANE

        @pl.when(hi)
        def _():
            pltpu.sync_copy(
                out2_v.at[:, pl.ds(half, cols_hi - half)],
                out_hbm.at[:, pl.ds(cb + half, cols_hi - half)])

        @pl.when(jnp.logical_and(jnp.logical_not(hi),
                                 jnp.logical_not(last)))
        def _():
            pltpu.sync_copy(
                out2_v.at[:, pl.ds(half, cols_lo - half)],
                out_hbm.at[:, pl.ds(cb + half, cols_lo - half)])

        cols_full = cols_last // LANE * LANE
        cols_edge = cols_last - cols_full

        @pl.when(last)
        def _():
            pltpu.sync_copy(
                out2_v.at[:, pl.ds(half, cols_full - half)],
                out_hbm.at[:, pl.ds(cb + half, cols_full - half)])
            pltpu.sync_copy(
                out2_v.at[:, pl.ds(cols_full, cols_edge)],
                out_hbm.at[:, pl.ds(cb + cols_full, cols_edge)])

        h_o.wait()

    return call


def kernel(list_scalars, src_vals, one_hot_dim):
    del one_hot_dim
    n = list_scalars.shape[0]
    out_t = _build_call(n)(list_scalars, src_vals)
    return out_t.T


# speedup vs baseline: 1.0074x; 1.0074x over previous
import functools

import jax
import jax.numpy as jnp
from jax import lax
from jax.experimental import pallas as pl
from jax.experimental.pallas import tpu as pltpu
from jax.experimental.pallas import tpu_sc as plsc

L = 16
D = 8
NC = 2
NS = 16
NW = NC * NS
LANE = 128


def _build_call(n):
    npad = -(-n // LANE) * LANE
    ntiles = npad // LANE
    tbase_w = ntiles // NW
    textra = ntiles - tbase_w * NW
    cols_hi = (tbase_w + 1) * LANE
    cols_lo = tbase_w * LANE
    shift_max = npad - n + LANE
    cols_buf = cols_hi + shift_max

    win = 2 * cols_hi
    win_buf = win + (npad - n)
    assert tbase_w % 2 == 0 and LANE // L == 8 and win // L % 2 == 0

    mesh = plsc.VectorSubcoreMesh(core_axis_name="c", subcore_axis_name="s")

    @functools.partial(
        pl.kernel,
        out_type=jax.ShapeDtypeStruct((D, n), jnp.float32),
        mesh=mesh,
        scratch_types=[
            pltpu.VMEM((win_buf,), jnp.int32),
            pltpu.VMEM((cols_buf,), jnp.float32),
            pltpu.VMEM((D, cols_hi), jnp.float32),
            pltpu.VMEM((L,), jnp.int32),
            pltpu.VMEM((NS * L,), jnp.int32),
            pltpu.VMEM_SHARED((NS * L,), jnp.int32),
            pltpu.SemaphoreType.DMA,
            pltpu.SemaphoreType.DMA,
            pltpu.SemaphoreType.DMA,
        ],
        compiler_params=pltpu.CompilerParams(needs_layout_passes=False),
    )
    def call(ls_hbm, sv_hbm, out_hbm,
             vals2_v, src_v, out2_v,
             minvec_v, allmins_v, mins_sh, sem_v, sem_s, sem_o):
        c = lax.axis_index("c")
        s = lax.axis_index("s")
        w = s * NC + c

        tb = tbase_w * w + jnp.minimum(w, textra)
        cb = tb * LANE
        base_s = jnp.minimum((tbase_w * (2 * s)
                              + jnp.minimum(2 * s, textra)) * LANE, n - win)
        shift2 = cb - base_s
        base2 = jnp.minimum(cb, n - cols_hi)
        shift = cb - base2
        hi = w < textra
        ngc4 = (tbase_w + jnp.where(hi, 1, 0)) * (LANE // L // 4)

        h_v = pltpu.async_copy(ls_hbm.at[pl.ds(base_s, win)],
                               vals2_v.at[pl.ds(0, win)], sem_v)
        h_s = pltpu.async_copy(sv_hbm.at[pl.ds(base2, cols_hi)],
                               src_v.at[pl.ds(0, cols_hi)], sem_s)

        h_v.wait()

        def mstep(i, m):
            for u in range(4):
                m = jnp.minimum(m, vals2_v[pl.ds(i * (4 * L) + u * L, L)])
            return m
        m = lax.fori_loop(0, win // L // 4, mstep,
                          jnp.full((L,), jnp.iinfo(jnp.int32).max,
                                   jnp.int32))
        minvec_v[...] = m
        pltpu.sync_copy(minvec_v, mins_sh.at[pl.ds(s * L, L)])
        plsc.subcore_barrier()
        pltpu.sync_copy(mins_sh, allmins_v)

        def mstep2(i, m):
            return jnp.minimum(m, allmins_v[pl.ds(i * L, L)])
        mall = lax.fori_loop(0, NS, mstep2,
                             jnp.full((L,), jnp.iinfo(jnp.int32).max,
                                      jnp.int32))
        gmin = mall[0]
        for j in range(1, L):
            gmin = jnp.minimum(gmin, mall[j])

        h_s.wait()
        fzero = jnp.zeros((L,), jnp.float32)
        lane = lax.iota(jnp.int32, L)

        def group(g):
            v = vals2_v[pl.ds(shift2 + g * L, L)] - gmin
            v = jnp.minimum(v, D - 1)
            v = jnp.maximum(v, 0)
            sv = src_v[pl.ds(shift + g * L, L)]
            for d in range(D):
                out2_v[d, pl.ds(g * L, L)] = fzero
            plsc.store_scatter(out2_v, [v, g * L + lane], sv)

        def wstep(i, carry):
            for u in range(4):
                group(4 * i + u)
            return carry

        ghalf = cols_lo // L // 2
        lax.fori_loop(0, ghalf // 4, wstep, 0)
        h_o = pltpu.async_copy(out2_v.at[:, pl.ds(0, cols_lo // 2)],
                               out_hbm.at[:, pl.ds(cb, cols_lo // 2)],
                               sem_o)
        lax.fori_loop(ghalf // 4, ngc4, wstep, 0)

        last = w >= NW - 1
        half = cols_lo // 2
        cols_last = n - (ntiles - tbase_w) * LANE

        @pl.when(hi)
        def _():
            pltpu.sync_copy(
                out2_v.at[:, pl.ds(half, cols_hi - half)],
                out_hbm.at[:, pl.ds(cb + half, cols_hi - half)])

        @pl.when(jnp.logical_and(jnp.logical_not(hi),
                                 jnp.logical_not(last)))
        def _():
            pltpu.sync_copy(
                out2_v.at[:, pl.ds(half, cols_lo - half)],
                out_hbm.at[:, pl.ds(cb + half, cols_lo - half)])

        cols_full = cols_last // LANE * LANE
        cols_edge = cols_last - cols_full

        @pl.when(last)
        def _():
            pltpu.sync_copy(
                out2_v.at[:, pl.ds(half, cols_full - half)],
                out_hbm.at[:, pl.ds(cb + half, cols_full - half)])
            pltpu.sync_copy(
                out2_v.at[:, pl.ds(cols_full, cols_edge)],
                out_hbm.at[:, pl.ds(cb + cols_full, cols_edge)])

        h_o.wait()

    return call


def kernel(list_scalars, src_vals, one_hot_dim):
    del one_hot_dim
    n = list_scalars.shape[0]
    out_t = _build_call(n)(list_scalars, src_vals)
    return out_t.T
